# baseline (speedup 1.0000x reference)
import jax
import jax.numpy as jnp
from jax import lax
from jax.experimental import pallas as pl
from jax.experimental.pallas import tpu as pltpu
from jax.experimental.pallas import tpu_sc as plsc

LINE_SEP = 5
N_LINES = 32
EMB_DIM = 1024
ROWS = 4
COLS = 8192
SCALE = EMB_DIM ** -0.5

NC, NS, L = 2, 16, 16
NW = NC * NS
CHUNK = (ROWS * COLS) // NW
SEGS = COLS // CHUNK
VPC = CHUNK // L
REP = 16
NSLOT = 2
Q = EMB_DIM // 4


def _scale_body(emb_ref, out_ref):
    out_ref[...] = emb_ref[...] * SCALE


def _scale_table(emb):
    return pl.pallas_call(
        _scale_body,
        out_shape=jax.ShapeDtypeStruct((N_LINES, EMB_DIM), jnp.float32),
    )(emb)


def _sc_body(x_hbm, emb_hbm, out_hbm, xall, tbl, idx, bnd, rep0, rep1, bb, sm,
             ssem0, ssem1):
    wid = lax.axis_index("s") * NC + lax.axis_index("c")
    row = wid // SEGS
    seg = wid % SEGS
    base = wid * CHUNK

    pltpu.sync_copy(emb_hbm, tbl)
    pltpu.sync_copy(x_hbm.at[pl.ds(row * COLS, COLS)], xall)

    def count_body(j, acc):
        v = xall[pl.ds(j * L, L)]
        return acc + jnp.where(v == LINE_SEP, 1, 0).astype(jnp.int32)

    acc = lax.fori_loop(0, seg * VPC, count_body, jnp.zeros((L,), jnp.int32))
    offset = jnp.sum(acc)

    iota = lax.iota(jnp.int32, L)
    for t in range(3):
        kk = iota + t * L
        bnd[pl.ds(t * L, L)] = jnp.where(kk <= offset, 0, CHUNK)

    def cum_body(j, carry):
        v = xall[pl.ds(seg * CHUNK + j * L, L)]
        sep = jnp.where(v == LINE_SEP, 1, 0).astype(jnp.int32)
        raw = carry + plsc.cumsum(sep)
        idx[pl.ds(j * L, L)] = jnp.minimum(raw, N_LINES - 1)
        pos = iota + j * L
        plsc.store_scatter(
            bnd, [jnp.minimum(raw, 47)], pos,
            mask=(sep > 0) & (raw <= N_LINES),
        )
        return raw[15]

    lax.fori_loop(0, VPC, cum_body, offset)

    v0 = bnd[pl.ds(0, L)]
    v1 = bnd[pl.ds(L, L)]
    for l in range(L):
        sm[l] = v0[l]
        sm[L + l] = v1[l]
    for l in range(L - 1):
        sm[32 + l] = v0[l + 1]
        sm[48 + l] = v1[l + 1]
    sm[47] = v1[0]
    sm[63] = CHUNK

    reps = (rep0, rep1)
    ssems = (ssem0, ssem1)

    def orow(pos):
        return pl.multiple_of(base + pos, 8)

    def big_piece(s, pos):
        pltpu.async_copy(
            reps[s], out_hbm.at[pl.ds(orow(pos), REP)], ssems[s]
        )

    def big_drain(s):
        pltpu.make_async_copy(
            reps[s], out_hbm.at[pl.ds(orow(0), REP)], ssems[s]
        ).wait()

    def fill_rows(dst, k, lo, hi):
        for q in range(4):
            regs = [tbl[k, pl.ds(q * Q + t * L, L)] for t in range(16)]

            def rbody(i, c, regs=regs, q=q, dst=dst):
                for t in range(16):
                    dst[i, pl.ds(q * Q + t * L, L)] = regs[t]
                return c

            lax.fori_loop(lo, hi, rbody, jnp.int32(0))

    def group_body(g, carry):
        p0, p1, prev_t = carry
        pend = [p0, p1]
        for s in range(NSLOT):
            k = g * NSLOT + s
            kc = jnp.minimum(k, N_LINES - 1)
            lo = sm[kc]
            hi = sm[32 + kc]
            ln = hi - lo
            al = (lo + 7) & ~7
            ah = hi & ~7
            span = ah - al
            n32 = jnp.where(span >= REP, (span + REP - 1) // REP, 0)
            t0 = lo >> 3
            mk_bb = (k < N_LINES) & (ln > 0) & ((lo & 7) != 0) & (t0 != prev_t)

            @pl.when((k < N_LINES) & (ln > 0))
            def _(s=s, kc=kc, lo=lo, hi=hi, al=al, ah=ah, span=span, n32=n32,
                  t0=t0, mk_bb=mk_bb, pending_s=pend[s]):
                def drain_body(i, c):
                    big_drain(s)
                    return c

                lax.fori_loop(0, pending_s, drain_body, jnp.int32(0))

                @pl.when(span > 0)
                def _():
                    fill_rows(reps[s], kc, 0, jnp.minimum(span, REP))

                    @pl.when(span >= REP)
                    def _():
                        def piece_body(i, c):
                            big_piece(s, al + i * REP)
                            return c

                        lax.fori_loop(0, n32 - 1, piece_body, jnp.int32(0))
                        big_piece(s, ah - REP)

                    s8 = span >> 3

                    @pl.when(s8 == 1)
                    def _():
                        pltpu.sync_copy(
                            reps[s].at[pl.ds(0, 8)],
                            out_hbm.at[pl.ds(orow(al), 8)],
                        )

                    @pl.when(s8 == 2)
                    def _():
                        pltpu.sync_copy(
                            reps[s].at[pl.ds(0, 16)],
                            out_hbm.at[pl.ds(orow(al), 16)],
                        )

                    @pl.when(s8 == 3)
                    def _():
                        pltpu.sync_copy(
                            reps[s].at[pl.ds(0, 16)],
                            out_hbm.at[pl.ds(orow(al), 16)],
                        )
                        pltpu.sync_copy(
                            reps[s].at[pl.ds(0, 16)],
                            out_hbm.at[pl.ds(orow(ah - 16), 16)],
                        )

                @pl.when(mk_bb)
                def _():
                    vk = idx[pl.ds(t0 * 8, L)]
                    for r in range(8):
                        fill_rows(bb, vk[r], r, r + 1)
                    pltpu.sync_copy(
                        bb, out_hbm.at[pl.ds(orow(t0 * 8), 8)]
                    )

            live = (k < N_LINES) & (ln > 0)
            pend[s] = jnp.where(live, jnp.where(span >= REP, n32, 0), pend[s])
            prev_t = jnp.where(mk_bb, t0, prev_t)
        return pend[0], pend[1], prev_t

    carry = lax.fori_loop(
        0, N_LINES // NSLOT, group_body,
        (jnp.int32(0), jnp.int32(0), jnp.int32(-1)),
    )
    for s in range(NSLOT):
        def drain_body(i, c, s=s):
            big_drain(s)
            return c

        lax.fori_loop(0, carry[s], drain_body, jnp.int32(0))


@jax.jit
def kernel(x, emb):
    x_flat = x.reshape(ROWS * COLS).astype(jnp.int32)
    emb_s = _scale_table(emb)
    mesh = plsc.VectorSubcoreMesh(
        core_axis_name="c", subcore_axis_name="s", num_cores=NC, num_subcores=NS
    )
    run = pl.kernel(
        _sc_body,
        out_type=jax.ShapeDtypeStruct((ROWS * COLS, EMB_DIM), jnp.float32),
        mesh=mesh,
        scratch_types=[
            pltpu.VMEM((COLS,), jnp.int32),
            pltpu.VMEM((N_LINES, EMB_DIM), jnp.float32),
            pltpu.VMEM((CHUNK + L,), jnp.int32),
            pltpu.VMEM((48,), jnp.int32),
            pltpu.VMEM((REP, EMB_DIM), jnp.float32),
            pltpu.VMEM((REP, EMB_DIM), jnp.float32),
            pltpu.VMEM((8, EMB_DIM), jnp.float32),
            pltpu.SMEM((64,), jnp.int32),
            pltpu.SemaphoreType.DMA,
            pltpu.SemaphoreType.DMA,
        ],
        compiler_params=pltpu.CompilerParams(needs_layout_passes=False),
    )
    out = run(x_flat, emb_s)
    return out.reshape(ROWS, COLS, EMB_DIM)


# speedup vs baseline: 1.0403x; 1.0403x over previous
import jax
import jax.numpy as jnp
from jax import lax
from jax.experimental import pallas as pl
from jax.experimental.pallas import tpu as pltpu
from jax.experimental.pallas import tpu_sc as plsc

LINE_SEP = 5
N_LINES = 32
EMB_DIM = 1024
ROWS = 4
COLS = 8192
SCALE = EMB_DIM ** -0.5

NC, NS, L = 2, 16, 16
NW = NC * NS
CHUNK = (ROWS * COLS) // NW
SEGS = COLS // CHUNK
VPC = CHUNK // L
REP = 32
NSLOT = 2
Q = EMB_DIM // 4


def _scale_body(emb_ref, out_ref):
    out_ref[...] = emb_ref[...] * SCALE


def _scale_table(emb):
    return pl.pallas_call(
        _scale_body,
        out_shape=jax.ShapeDtypeStruct((N_LINES, EMB_DIM), jnp.float32),
    )(emb)


def _sc_body(x_hbm, emb_hbm, out_hbm, xall, tbl, idx, bnd, rep0, rep1, bb, sm,
             ssem0, ssem1, tsem):
    wid = lax.axis_index("s") * NC + lax.axis_index("c")
    row = wid // SEGS
    seg = wid % SEGS
    base = wid * CHUNK

    tcopy = pltpu.async_copy(emb_hbm, tbl, tsem)
    pltpu.sync_copy(x_hbm.at[pl.ds(row * COLS, COLS)], xall)

    def count_body(j, acc):
        v = xall[pl.ds(j * L, L)]
        return acc + jnp.where(v == LINE_SEP, 1, 0).astype(jnp.int32)

    acc = lax.fori_loop(0, seg * VPC, count_body, jnp.zeros((L,), jnp.int32))
    offset = jnp.sum(acc)

    iota = lax.iota(jnp.int32, L)
    for t in range(3):
        kk = iota + t * L
        bnd[pl.ds(t * L, L)] = jnp.where(kk <= offset, 0, CHUNK)

    def cum_body(j, carry):
        v = xall[pl.ds(seg * CHUNK + j * L, L)]
        sep = jnp.where(v == LINE_SEP, 1, 0).astype(jnp.int32)
        raw = carry + plsc.cumsum(sep)
        idx[pl.ds(j * L, L)] = jnp.minimum(raw, N_LINES - 1)
        pos = iota + j * L
        plsc.store_scatter(
            bnd, [jnp.minimum(raw, 47)], pos,
            mask=(sep > 0) & (raw <= N_LINES),
        )
        return carry + jnp.sum(sep)

    lax.fori_loop(0, VPC, cum_body, offset)

    v0 = bnd[pl.ds(0, L)]
    v1 = bnd[pl.ds(L, L)]
    for l in range(L):
        sm[l] = v0[l]
        sm[L + l] = v1[l]
    for l in range(L - 1):
        sm[32 + l] = v0[l + 1]
        sm[48 + l] = v1[l + 1]
    sm[47] = v1[0]
    sm[63] = CHUNK

    tcopy.wait()
    reps = (rep0, rep1)
    ssems = (ssem0, ssem1)

    def orow(pos):
        return pl.multiple_of(base + pos, 8)

    def big_piece(s, pos):
        pltpu.async_copy(
            reps[s], out_hbm.at[pl.ds(orow(pos), REP)], ssems[s]
        )

    def big_drain(s):
        pltpu.make_async_copy(
            reps[s], out_hbm.at[pl.ds(orow(0), REP)], ssems[s]
        ).wait()

    def fill_rows(dst, k, lo, hi):
        for q in range(4):
            regs = [tbl[k, pl.ds(q * Q + t * L, L)] for t in range(16)]

            def rbody(i, c, regs=regs, q=q, dst=dst):
                for t in range(16):
                    dst[i, pl.ds(q * Q + t * L, L)] = regs[t]
                return c

            lax.fori_loop(lo, hi, rbody, jnp.int32(0))

    def group_body(g, carry):
        p0, p1, prev_t = carry
        pend = [p0, p1]
        for s in range(NSLOT):
            k = g * NSLOT + s
            kc = jnp.minimum(k, N_LINES - 1)
            lo = sm[kc]
            hi = sm[32 + kc]
            ln = hi - lo
            al = (lo + 7) & ~7
            ah = hi & ~7
            span = ah - al
            n32 = jnp.where(span >= REP, (span + REP - 1) // REP, 0)
            t0 = lo >> 3
            mk_bb = (k < N_LINES) & (ln > 0) & ((lo & 7) != 0) & (t0 != prev_t)

            @pl.when((k < N_LINES) & (ln > 0))
            def _(s=s, kc=kc, lo=lo, hi=hi, al=al, ah=ah, span=span, n32=n32,
                  t0=t0, mk_bb=mk_bb, pending_s=pend[s]):
                def drain_body(i, c):
                    big_drain(s)
                    return c

                lax.fori_loop(0, pending_s, drain_body, jnp.int32(0))

                @pl.when(span > 0)
                def _():
                    fill_rows(reps[s], kc, 0, jnp.minimum(span, REP))

                    @pl.when(span >= REP)
                    def _():
                        def piece_body(i, c):
                            big_piece(s, al + i * REP)
                            return c

                        lax.fori_loop(0, n32 - 1, piece_body, jnp.int32(0))
                        big_piece(s, ah - REP)

                    s8 = span >> 3

                    @pl.when(s8 == 1)
                    def _():
                        pltpu.sync_copy(
                            reps[s].at[pl.ds(0, 8)],
                            out_hbm.at[pl.ds(orow(al), 8)],
                        )

                    @pl.when(s8 == 2)
                    def _():
                        pltpu.sync_copy(
                            reps[s].at[pl.ds(0, 16)],
                            out_hbm.at[pl.ds(orow(al), 16)],
                        )

                    @pl.when(s8 == 3)
                    def _():
                        pltpu.sync_copy(
                            reps[s].at[pl.ds(0, 16)],
                            out_hbm.at[pl.ds(orow(al), 16)],
                        )
                        pltpu.sync_copy(
                            reps[s].at[pl.ds(0, 16)],
                            out_hbm.at[pl.ds(orow(ah - 16), 16)],
                        )

                @pl.when(mk_bb)
                def _():
                    vk = idx[pl.ds(t0 * 8, L)]
                    for r in range(8):
                        fill_rows(bb, vk[r], r, r + 1)
                    pltpu.sync_copy(
                        bb, out_hbm.at[pl.ds(orow(t0 * 8), 8)]
                    )

            live = (k < N_LINES) & (ln > 0)
            pend[s] = jnp.where(live, jnp.where(span >= REP, n32, 0), pend[s])
            prev_t = jnp.where(mk_bb, t0, prev_t)
        return pend[0], pend[1], prev_t

    carry = lax.fori_loop(
        0, N_LINES // NSLOT, group_body,
        (jnp.int32(0), jnp.int32(0), jnp.int32(-1)),
    )
    for s in range(NSLOT):
        def drain_body(i, c, s=s):
            big_drain(s)
            return c

        lax.fori_loop(0, carry[s], drain_body, jnp.int32(0))


@jax.jit
def kernel(x, emb):
    x_flat = x.reshape(ROWS * COLS).astype(jnp.int32)
    emb_s = _scale_table(emb)
    mesh = plsc.VectorSubcoreMesh(
        core_axis_name="c", subcore_axis_name="s", num_cores=NC, num_subcores=NS
    )
    run = pl.kernel(
        _sc_body,
        out_type=jax.ShapeDtypeStruct((ROWS * COLS, EMB_DIM), jnp.float32),
        mesh=mesh,
        scratch_types=[
            pltpu.VMEM((COLS,), jnp.int32),
            pltpu.VMEM((N_LINES, EMB_DIM), jnp.float32),
            pltpu.VMEM((CHUNK + L,), jnp.int32),
            pltpu.VMEM((48,), jnp.int32),
            pltpu.VMEM((REP, EMB_DIM), jnp.float32),
            pltpu.VMEM((REP, EMB_DIM), jnp.float32),
            pltpu.VMEM((8, EMB_DIM), jnp.float32),
            pltpu.SMEM((64,), jnp.int32),
            pltpu.SemaphoreType.DMA,
            pltpu.SemaphoreType.DMA,
            pltpu.SemaphoreType.DMA,
        ],
        compiler_params=pltpu.CompilerParams(needs_layout_passes=False),
    )
    out = run(x_flat, emb_s)
    return out.reshape(ROWS, COLS, EMB_DIM)


# speedup vs baseline: 1.0493x; 1.0086x over previous
import jax
import jax.numpy as jnp
from jax import lax
from jax.experimental import pallas as pl
from jax.experimental.pallas import tpu as pltpu
from jax.experimental.pallas import tpu_sc as plsc

LINE_SEP = 5
N_LINES = 32
EMB_DIM = 1024
ROWS = 4
COLS = 8192
SCALE = EMB_DIM ** -0.5

NC, NS, L = 2, 16, 16
NW = NC * NS
CHUNK = (ROWS * COLS) // NW
SEGS = COLS // CHUNK
VPC = CHUNK // L
REP = 32
NSLOT = 2
Q = EMB_DIM // 4


def _scale_body(emb_ref, out_ref):
    out_ref[...] = emb_ref[...] * SCALE


def _scale_table(emb):
    return pl.pallas_call(
        _scale_body,
        out_shape=jax.ShapeDtypeStruct((N_LINES, EMB_DIM), jnp.float32),
    )(emb)


def _sc_body(x_hbm, emb_hbm, out_hbm, xall, tbl, idx, bnd, rep0, rep1, bb, sm,
             ssem0, ssem1, tsem):
    wid = lax.axis_index("s") * NC + lax.axis_index("c")
    row = wid // SEGS
    seg = wid % SEGS
    base = wid * CHUNK

    tcopy = pltpu.async_copy(emb_hbm, tbl, tsem)
    pltpu.sync_copy(x_hbm.at[pl.ds(row * COLS, COLS)], xall)

    def count_body(j, acc):
        for u in range(4):
            v = xall[pl.ds(j * 4 * L + u * L, L)]
            acc = acc + jnp.where(v == LINE_SEP, 1, 0).astype(jnp.int32)
        return acc

    acc = lax.fori_loop(0, seg * (VPC // 4), count_body,
                        jnp.zeros((L,), jnp.int32))
    offset = jnp.sum(acc)

    iota = lax.iota(jnp.int32, L)
    for t in range(3):
        kk = iota + t * L
        bnd[pl.ds(t * L, L)] = jnp.where(kk <= offset, 0, CHUNK)

    def cum_body(j, carry):
        v = xall[pl.ds(seg * CHUNK + j * L, L)]
        sep = jnp.where(v == LINE_SEP, 1, 0).astype(jnp.int32)
        raw = carry + plsc.cumsum(sep)
        idx[pl.ds(j * L, L)] = jnp.minimum(raw, N_LINES - 1)
        pos = iota + j * L
        plsc.store_scatter(
            bnd, [jnp.minimum(raw, 47)], pos,
            mask=(sep > 0) & (raw <= N_LINES),
        )
        return carry + jnp.sum(sep)

    lax.fori_loop(0, VPC, cum_body, offset)

    v0 = bnd[pl.ds(0, L)]
    v1 = bnd[pl.ds(L, L)]
    for l in range(L):
        sm[l] = v0[l]
        sm[L + l] = v1[l]
    for l in range(L - 1):
        sm[32 + l] = v0[l + 1]
        sm[48 + l] = v1[l + 1]
    sm[47] = v1[0]
    sm[63] = CHUNK

    tcopy.wait()
    reps = (rep0, rep1)
    ssems = (ssem0, ssem1)

    def orow(pos):
        return pl.multiple_of(base + pos, 8)

    def big_piece(s, pos):
        pltpu.async_copy(
            reps[s], out_hbm.at[pl.ds(orow(pos), REP)], ssems[s]
        )

    def big_drain(s):
        pltpu.make_async_copy(
            reps[s], out_hbm.at[pl.ds(orow(0), REP)], ssems[s]
        ).wait()

    def fill_rows(dst, k, lo, hi):
        for q in range(4):
            regs = [tbl[k, pl.ds(q * Q + t * L, L)] for t in range(16)]

            def rbody(i, c, regs=regs, q=q, dst=dst):
                for t in range(16):
                    dst[i, pl.ds(q * Q + t * L, L)] = regs[t]
                return c

            lax.fori_loop(lo, hi, rbody, jnp.int32(0))

    def group_body(g, carry):
        p0, p1, prev_t = carry
        pend = [p0, p1]
        for s in range(NSLOT):
            k = g * NSLOT + s
            kc = jnp.minimum(k, N_LINES - 1)
            lo = sm[kc]
            hi = sm[32 + kc]
            ln = hi - lo
            al = (lo + 7) & ~7
            ah = hi & ~7
            span = ah - al
            n32 = jnp.where(span >= REP, (span + REP - 1) // REP, 0)
            t0 = lo >> 3
            mk_bb = (k < N_LINES) & (ln > 0) & ((lo & 7) != 0) & (t0 != prev_t)

            @pl.when((k < N_LINES) & (ln > 0))
            def _(s=s, kc=kc, lo=lo, hi=hi, al=al, ah=ah, span=span, n32=n32,
                  t0=t0, mk_bb=mk_bb, pending_s=pend[s]):
                def drain_body(i, c):
                    big_drain(s)
                    return c

                lax.fori_loop(0, pending_s, drain_body, jnp.int32(0))

                @pl.when(span > 0)
                def _():
                    fill_rows(reps[s], kc, 0, jnp.minimum(span, REP))

                    @pl.when(span >= REP)
                    def _():
                        def piece_body(i, c):
                            big_piece(s, al + i * REP)
                            return c

                        lax.fori_loop(0, n32 - 1, piece_body, jnp.int32(0))
                        big_piece(s, ah - REP)

                    s8 = span >> 3

                    @pl.when(s8 == 1)
                    def _():
                        pltpu.sync_copy(
                            reps[s].at[pl.ds(0, 8)],
                            out_hbm.at[pl.ds(orow(al), 8)],
                        )

                    @pl.when(s8 == 2)
                    def _():
                        pltpu.sync_copy(
                            reps[s].at[pl.ds(0, 16)],
                            out_hbm.at[pl.ds(orow(al), 16)],
                        )

                    @pl.when(s8 == 3)
                    def _():
                        pltpu.sync_copy(
                            reps[s].at[pl.ds(0, 16)],
                            out_hbm.at[pl.ds(orow(al), 16)],
                        )
                        pltpu.sync_copy(
                            reps[s].at[pl.ds(0, 16)],
                            out_hbm.at[pl.ds(orow(ah - 16), 16)],
                        )

                @pl.when(mk_bb)
                def _():
                    vk = idx[pl.ds(t0 * 8, L)]
                    for r in range(8):
                        fill_rows(bb, vk[r], r, r + 1)
                    pltpu.sync_copy(
                        bb, out_hbm.at[pl.ds(orow(t0 * 8), 8)]
                    )

            live = (k < N_LINES) & (ln > 0)
            pend[s] = jnp.where(live, jnp.where(span >= REP, n32, 0), pend[s])
            prev_t = jnp.where(mk_bb, t0, prev_t)
        return pend[0], pend[1], prev_t

    carry = lax.fori_loop(
        0, N_LINES // NSLOT, group_body,
        (jnp.int32(0), jnp.int32(0), jnp.int32(-1)),
    )
    for s in range(NSLOT):
        def drain_body(i, c, s=s):
            big_drain(s)
            return c

        lax.fori_loop(0, carry[s], drain_body, jnp.int32(0))


@jax.jit
def kernel(x, emb):
    x_flat = x.reshape(ROWS * COLS).astype(jnp.int32)
    emb_s = _scale_table(emb)
    mesh = plsc.VectorSubcoreMesh(
        core_axis_name="c", subcore_axis_name="s", num_cores=NC, num_subcores=NS
    )
    run = pl.kernel(
        _sc_body,
        out_type=jax.ShapeDtypeStruct((ROWS * COLS, EMB_DIM), jnp.float32),
        mesh=mesh,
        scratch_types=[
            pltpu.VMEM((COLS,), jnp.int32),
            pltpu.VMEM((N_LINES, EMB_DIM), jnp.float32),
            pltpu.VMEM((CHUNK + L,), jnp.int32),
            pltpu.VMEM((48,), jnp.int32),
            pltpu.VMEM((REP, EMB_DIM), jnp.float32),
            pltpu.VMEM((REP, EMB_DIM), jnp.float32),
            pltpu.VMEM((8, EMB_DIM), jnp.float32),
            pltpu.SMEM((64,), jnp.int32),
            pltpu.SemaphoreType.DMA,
            pltpu.SemaphoreType.DMA,
            pltpu.SemaphoreType.DMA,
        ],
        compiler_params=pltpu.CompilerParams(needs_layout_passes=False),
    )
    out = run(x_flat, emb_s)
    return out.reshape(ROWS, COLS, EMB_DIM)
